# baseline (speedup 1.0000x reference)
import jax
import jax.numpy as jnp
from jax.experimental import pallas as pl
from jax.experimental.pallas import tpu as pltpu

_N = 325
_IN = 2
_H = 64
_M = 3
_T = 12
_B = 64
_NP = 352
_BH = _B * _H
_P2 = 2 * _H
_BIN = _B * _IN
_NQT = 4
_BQ = _B // _NQT
_QW = _BQ * _H
_QR = _NP * _BQ // 2

_BF = jnp.bfloat16
_F32 = jnp.float32


def _step_body(xg_ref, xc_ref, A_ref,
               wg0_ref, bg0_ref, wc0_ref, bc0_ref,
               wg1_ref, bg1_ref, wc1_ref, bc1_ref,
               out_ref, h0f_ref,
               h0_s, h1_s, c01_s, c02_s):
    t = pl.program_id(0)

    @pl.when(t == 0)
    def _init():
        zf = jnp.zeros((_NP, _BH), _F32)
        zb = jnp.zeros((_NQT, _QR, _P2), _BF)
        h0_s[...] = zf
        h1_s[...] = zf
        c01_s[...] = zb
        c02_s[...] = zb

    A = A_ref[...]

    def dot(a, b):
        return jnp.dot(a, b, preferred_element_type=_F32)

    def cheb(xb, xf):
        x1f = dot(A, xb)
        x2f = 2.0 * dot(A, x1f.astype(_BF)) - xf
        return x1f, x2f

    def pairq(x):
        return x.reshape(_QR, _P2)

    def unpairq(x):
        return x.reshape(_NP, _QW)

    def proj(pieces, W):
        outs = []
        for off in (0, _H):
            halves = [p[:, off:off + _H] for p in pieces]
            acc = None
            i = 0
            while i < len(halves):
                j = min(i + 2, len(halves))
                blk = (halves[i] if j == i + 1
                       else jnp.concatenate(halves[i:j], axis=1))
                d = dot(blk, W[i * _H:j * _H])
                acc = d if acc is None else acc + d
                i = j
            outs.append(acc)
        return outs[0], outs[1]

    for q in range(_NQT):
        qs = slice(q * _QW, (q + 1) * _QW)

        xg_ev = xg_ref[0, 2 * q].astype(_F32)
        xg_od = xg_ref[0, 2 * q + 1].astype(_F32)
        xcq = xc_ref[0, q].astype(_F32)

        h0f = h0_s[:, qs]
        h0pf = pairq(h0f)
        h0pb = h0pf.astype(_BF)
        g_ev, g_od = proj([h0pb, c01_s[q], c02_s[q]], wg0_ref[...])
        val_ev = jax.nn.sigmoid(g_ev + xg_ev + bg0_ref[...])
        val_od = jax.nn.sigmoid(g_od + xg_od + bg0_ref[...])
        r = jnp.concatenate([val_ev[:, :_H], val_od[:, :_H]], axis=1)
        u = jnp.concatenate([val_ev[:, _H:], val_od[:, _H:]], axis=1)
        rhf = r * h0pf
        rhd = unpairq(rhf)
        rh1f, rh2f = cheb(rhd.astype(_BF), rhd)
        c_ev, c_od = proj([rhf.astype(_BF),
                           pairq(rh1f).astype(_BF),
                           pairq(rh2f).astype(_BF)], wc0_ref[...])
        c_ev = jnp.tanh(c_ev + xcq[:, :_H] + bc0_ref[...])
        c_od = jnp.tanh(c_od + xcq[:, _H:] + bc0_ref[...])
        c = jnp.concatenate([c_ev, c_od], axis=1)
        h0n = u * h0pf + (1.0 - u) * c
        h0nd = unpairq(h0n)
        h0_s[:, qs] = h0nd
        hn1f, hn2f = cheb(h0nd.astype(_BF), h0nd)
        h0np = h0n.astype(_BF)
        hn1p = pairq(hn1f).astype(_BF)
        hn2p = pairq(hn2f).astype(_BF)
        c01_s[q] = hn1p
        c02_s[q] = hn2p

        h1f = h1_s[:, qs]
        c11f, c12f = cheb(h1f.astype(_BF), h1f)
        h1pf = pairq(h1f)
        g1_ev, g1_od = proj([h1pf.astype(_BF),
                             pairq(c11f).astype(_BF),
                             pairq(c12f).astype(_BF),
                             h0np, hn1p, hn2p], wg1_ref[...])
        val1_ev = jax.nn.sigmoid(g1_ev + bg1_ref[...])
        val1_od = jax.nn.sigmoid(g1_od + bg1_ref[...])
        r1 = jnp.concatenate([val1_ev[:, :_H], val1_od[:, :_H]], axis=1)
        u1 = jnp.concatenate([val1_ev[:, _H:], val1_od[:, _H:]], axis=1)
        r1h1 = r1 * h1pf
        r1d = unpairq(r1h1)
        rr1f, rr2f = cheb(r1d.astype(_BF), r1d)
        c1_ev, c1_od = proj([r1h1.astype(_BF),
                             pairq(rr1f).astype(_BF),
                             pairq(rr2f).astype(_BF),
                             h0np, hn1p, hn2p], wc1_ref[...])
        c1_ev = jnp.tanh(c1_ev + bc1_ref[...])
        c1_od = jnp.tanh(c1_od + bc1_ref[...])
        c1 = jnp.concatenate([c1_ev, c1_od], axis=1)
        h1n = u1 * h1pf + (1.0 - u1) * c1
        h1nd = unpairq(h1n)
        h1_s[:, qs] = h1nd
        out_ref[0, :, qs] = h1nd.astype(_BF)

        @pl.when(t == _T - 1)
        def _fin():
            h0f_ref[:, qs] = h0nd.astype(_BF)


def _state_w(W, inx, out):
    Wr = W.reshape(inx + _H, _M, out)
    return Wr[inx:].transpose(1, 0, 2).reshape(_M * _H, out)


def _input_w(W, out):
    Wr = W.reshape(2 * _H, _M, out)
    return Wr[:_H].transpose(1, 0, 2).reshape(_M * _H, out)


@jax.jit
def kernel(inputs, initial_hidden_state, supports, Wg0, bg0, Wc0, bc0,
           Wg1, bg1, Wc1, bc1):
    A = supports[0]
    A_pf = jnp.zeros((_NP, _NP), _F32).at[:_N, :_N].set(A)
    A_p = A_pf.astype(_BF)
    x0 = jnp.transpose(inputs, (0, 2, 1, 3))
    x0 = jnp.pad(x0, ((0, 0), (0, _NP - _N), (0, 0), (0, 0)))

    x1 = jnp.einsum('nm,tmbj->tnbj', A_pf, x0)
    x2 = 2.0 * jnp.einsum('nm,tmbj->tnbj', A_pf, x1) - x0
    xm = jnp.stack([x0, x1, x2], axis=0)

    def _xmix(W, out):
        Wx = W.reshape(_IN + _H, _M, out)[:_IN].transpose(1, 0, 2)
        return jnp.einsum('mtnbj,mjo->tnbo', xm, Wx)

    xg_raw = _xmix(Wg0, 2 * _H)
    xc_raw = _xmix(Wc0, _H)
    xg = (xg_raw.reshape(_T, _NP, _NQT, _BQ // 2, 2, 2 * _H)
          .transpose(0, 2, 4, 1, 3, 5)
          .reshape(_T, 2 * _NQT, _QR, _P2).astype(_BF))
    xc = (xc_raw.reshape(_T, _NP, _NQT, _BQ // 2, 2, _H)
          .transpose(0, 2, 1, 3, 4, 5)
          .reshape(_T, _NQT, _QR, _P2).astype(_BF))

    wg0 = _state_w(Wg0, _IN, 2 * _H).astype(_BF)
    wc0 = _state_w(Wc0, _IN, _H).astype(_BF)
    wg1 = jnp.concatenate([_state_w(Wg1, _H, 2 * _H),
                           _input_w(Wg1, 2 * _H)], axis=0).astype(_BF)
    wc1 = jnp.concatenate([_state_w(Wc1, _H, _H),
                           _input_w(Wc1, _H)], axis=0).astype(_BF)
    bg0r = bg0.reshape(1, 2 * _H)
    bc0r = bc0.reshape(1, _H)
    bg1r = bg1.reshape(1, 2 * _H)
    bc1r = bc1.reshape(1, _H)

    out1, h0fin = pl.pallas_call(
        _step_body,
        grid=(_T,),
        in_specs=[
            pl.BlockSpec((1, 2 * _NQT, _QR, _P2), lambda t: (t, 0, 0, 0)),
            pl.BlockSpec((1, _NQT, _QR, _P2), lambda t: (t, 0, 0, 0)),
            pl.BlockSpec((_NP, _NP), lambda t: (0, 0)),
            pl.BlockSpec(wg0.shape, lambda t: (0, 0)),
            pl.BlockSpec(bg0r.shape, lambda t: (0, 0)),
            pl.BlockSpec(wc0.shape, lambda t: (0, 0)),
            pl.BlockSpec(bc0r.shape, lambda t: (0, 0)),
            pl.BlockSpec(wg1.shape, lambda t: (0, 0)),
            pl.BlockSpec(bg1r.shape, lambda t: (0, 0)),
            pl.BlockSpec(wc1.shape, lambda t: (0, 0)),
            pl.BlockSpec(bc1r.shape, lambda t: (0, 0)),
        ],
        out_specs=[
            pl.BlockSpec((1, _NP, _BH), lambda t: (t, 0, 0)),
            pl.BlockSpec((_NP, _BH), lambda t: (0, 0)),
        ],
        out_shape=[
            jax.ShapeDtypeStruct((_T, _NP, _BH), _BF),
            jax.ShapeDtypeStruct((_NP, _BH), _BF),
        ],
        scratch_shapes=[
            pltpu.VMEM((_NP, _BH), _F32),
            pltpu.VMEM((_NP, _BH), _F32),
            pltpu.VMEM((_NQT, _QR, _P2), _BF),
            pltpu.VMEM((_NQT, _QR, _P2), _BF),
        ],
        compiler_params=pltpu.CompilerParams(
            dimension_semantics=("arbitrary",),
            vmem_limit_bytes=100 * 1024 * 1024),
    )(xg, xc, A_p, wg0, bg0r, wc0, bc0r, wg1, bg1r, wc1, bc1r)

    cur = (out1[:, :_N]
           .astype(_F32)
           .reshape(_T, _N, _B, _H)
           .transpose(0, 2, 1, 3)
           .reshape(_T, _B, _N * _H))
    h0out = (h0fin[:_N]
             .astype(_F32)
             .reshape(_N, _B, _H)
             .transpose(1, 0, 2)
             .reshape(_B, _N * _H))
    return jnp.stack([h0out, cur[-1]], axis=0), cur


# speedup vs baseline: 1.2666x; 1.2666x over previous
import jax
import jax.numpy as jnp
from jax.experimental import pallas as pl
from jax.experimental.pallas import tpu as pltpu

_N = 325
_IN = 2
_H = 64
_M = 3
_T = 12
_B = 64
_NP = 352
_BH = _B * _H
_P2 = 2 * _H
_BIN = _B * _IN
_NQT = 2
_BQ = _B // _NQT
_QW = _BQ * _H
_QR = _NP * _BQ // 2

_BF = jnp.bfloat16
_F32 = jnp.float32


def _step_body(x_ref, A_ref, gg_ref, gc_ref,
               wg0_ref, bg0_ref, wc0_ref, bc0_ref,
               wg1_ref, bg1_ref, wc1_ref, bc1_ref,
               out_ref, h0f_ref,
               h0_s, h1_s, c01_s, c02_s):
    t = pl.program_id(0)

    @pl.when(t == 0)
    def _init():
        zf = jnp.zeros((_NP, _BH), _F32)
        zb = jnp.zeros((_NQT, _QR, _P2), _BF)
        h0_s[...] = zf
        h1_s[...] = zf
        c01_s[...] = zb
        c02_s[...] = zb

    A = A_ref[...]

    def dot(a, b):
        return jnp.dot(a, b, preferred_element_type=_F32)

    def cheb(xb, xf):
        x1f = dot(A, xb)
        x2f = 2.0 * dot(A, x1f.astype(_BF)) - xf
        return x1f, x2f

    def pairq(x):
        return x.reshape(_QR, _P2)

    def unpairq(x):
        return x.reshape(_NP, _QW)

    def proj(pieces, W):
        outs = []
        for off in (0, _H):
            halves = [p[:, off:off + _H] for p in pieces]
            blk = (halves[0] if len(halves) == 1
                   else jnp.concatenate(halves, axis=1))
            outs.append(dot(blk, W))
        return outs[0], outs[1]

    x0b = x_ref[0]
    x1f = dot(A, x0b)
    x2f = 2.0 * dot(A, x1f.astype(_BF)) - x0b.astype(_F32)
    xc3 = jnp.concatenate([x0b, x1f.astype(_BF), x2f.astype(_BF)], axis=1)

    for q in range(_NQT):
        qs = slice(q * _QW, (q + 1) * _QW)

        xg_ev = pairq(dot(xc3, gg_ref[:, (2 * q) * _QW:(2 * q + 1) * _QW]))
        xg_od = pairq(dot(xc3, gg_ref[:, (2 * q + 1) * _QW:(2 * q + 2) * _QW]))
        xcq = pairq(dot(xc3, gc_ref[:, q * _QW:(q + 1) * _QW]))

        h0f = h0_s[:, qs]
        h0pf = pairq(h0f)
        h0pb = h0pf.astype(_BF)
        g_ev, g_od = proj([h0pb, c01_s[q], c02_s[q]], wg0_ref[...])
        val_ev = jax.nn.sigmoid(g_ev + xg_ev + bg0_ref[...])
        val_od = jax.nn.sigmoid(g_od + xg_od + bg0_ref[...])
        r = jnp.concatenate([val_ev[:, :_H], val_od[:, :_H]], axis=1)
        u = jnp.concatenate([val_ev[:, _H:], val_od[:, _H:]], axis=1)
        rhf = r * h0pf
        rhd = unpairq(rhf)
        rh1f, rh2f = cheb(rhd.astype(_BF), rhd)
        c_ev, c_od = proj([rhf.astype(_BF),
                           pairq(rh1f).astype(_BF),
                           pairq(rh2f).astype(_BF)], wc0_ref[...])
        c_ev = jnp.tanh(c_ev + xcq[:, :_H] + bc0_ref[...])
        c_od = jnp.tanh(c_od + xcq[:, _H:] + bc0_ref[...])
        c = jnp.concatenate([c_ev, c_od], axis=1)
        h0n = u * h0pf + (1.0 - u) * c
        h0nd = unpairq(h0n)
        h0_s[:, qs] = h0nd
        hn1f, hn2f = cheb(h0nd.astype(_BF), h0nd)
        h0np = h0n.astype(_BF)
        hn1p = pairq(hn1f).astype(_BF)
        hn2p = pairq(hn2f).astype(_BF)
        c01_s[q] = hn1p
        c02_s[q] = hn2p

        h1f = h1_s[:, qs]
        c11f, c12f = cheb(h1f.astype(_BF), h1f)
        h1pf = pairq(h1f)
        g1_ev, g1_od = proj([h1pf.astype(_BF),
                             pairq(c11f).astype(_BF),
                             pairq(c12f).astype(_BF),
                             h0np, hn1p, hn2p], wg1_ref[...])
        val1_ev = jax.nn.sigmoid(g1_ev + bg1_ref[...])
        val1_od = jax.nn.sigmoid(g1_od + bg1_ref[...])
        r1 = jnp.concatenate([val1_ev[:, :_H], val1_od[:, :_H]], axis=1)
        u1 = jnp.concatenate([val1_ev[:, _H:], val1_od[:, _H:]], axis=1)
        r1h1 = r1 * h1pf
        r1d = unpairq(r1h1)
        rr1f, rr2f = cheb(r1d.astype(_BF), r1d)
        c1_ev, c1_od = proj([r1h1.astype(_BF),
                             pairq(rr1f).astype(_BF),
                             pairq(rr2f).astype(_BF),
                             h0np, hn1p, hn2p], wc1_ref[...])
        c1_ev = jnp.tanh(c1_ev + bc1_ref[...])
        c1_od = jnp.tanh(c1_od + bc1_ref[...])
        c1 = jnp.concatenate([c1_ev, c1_od], axis=1)
        h1n = u1 * h1pf + (1.0 - u1) * c1
        h1nd = unpairq(h1n)
        h1_s[:, qs] = h1nd
        out_ref[0, :, qs] = h1nd.astype(_BF)

        @pl.when(t == _T - 1)
        def _fin():
            h0f_ref[:, qs] = h0nd.astype(_BF)


def _state_w(W, inx, out):
    Wr = W.reshape(inx + _H, _M, out)
    return Wr[inx:].transpose(1, 0, 2).reshape(_M * _H, out)


def _input_w(W, out):
    Wr = W.reshape(2 * _H, _M, out)
    return Wr[:_H].transpose(1, 0, 2).reshape(_M * _H, out)


def _premix(W, out, b_order):
    Wx = W.reshape(_IN + _H, _M, out)[:_IN].transpose(1, 0, 2)
    eye = jnp.eye(_B, dtype=_F32)
    G = jnp.einsum('mjo,bc->mbjco', Wx, eye)
    G = G[:, :, :, jnp.array(b_order), :]
    return G.reshape(_M * _B * _IN, _B * out)


@jax.jit
def kernel(inputs, initial_hidden_state, supports, Wg0, bg0, Wc0, bc0,
           Wg1, bg1, Wc1, bc1):
    A = supports[0]
    A_p = jnp.zeros((_NP, _NP), _F32).at[:_N, :_N].set(A).astype(_BF)
    x = jnp.transpose(inputs, (0, 2, 1, 3))
    x = jnp.pad(x, ((0, 0), (0, _NP - _N), (0, 0), (0, 0)))
    x = x.reshape(_T, _NP, _BIN).astype(_BF)

    bo_gate = [_BQ * q + 2 * k + p
               for q in range(_NQT) for p in range(2) for k in range(_BQ // 2)]
    bo_cand = [_BQ * q + 2 * k + p
               for q in range(_NQT) for k in range(_BQ // 2) for p in range(2)]
    gg = _premix(Wg0, 2 * _H, bo_gate).astype(_BF)
    gc = _premix(Wc0, _H, bo_cand).astype(_BF)
    wg0 = _state_w(Wg0, _IN, 2 * _H).astype(_BF)
    wc0 = _state_w(Wc0, _IN, _H).astype(_BF)
    wg1 = jnp.concatenate([_state_w(Wg1, _H, 2 * _H),
                           _input_w(Wg1, 2 * _H)], axis=0).astype(_BF)
    wc1 = jnp.concatenate([_state_w(Wc1, _H, _H),
                           _input_w(Wc1, _H)], axis=0).astype(_BF)
    bg0r = bg0.reshape(1, 2 * _H)
    bc0r = bc0.reshape(1, _H)
    bg1r = bg1.reshape(1, 2 * _H)
    bc1r = bc1.reshape(1, _H)

    out1, h0fin = pl.pallas_call(
        _step_body,
        grid=(_T,),
        in_specs=[
            pl.BlockSpec((1, _NP, _BIN), lambda t: (t, 0, 0)),
            pl.BlockSpec((_NP, _NP), lambda t: (0, 0)),
            pl.BlockSpec(gg.shape, lambda t: (0, 0)),
            pl.BlockSpec(gc.shape, lambda t: (0, 0)),
            pl.BlockSpec(wg0.shape, lambda t: (0, 0)),
            pl.BlockSpec(bg0r.shape, lambda t: (0, 0)),
            pl.BlockSpec(wc0.shape, lambda t: (0, 0)),
            pl.BlockSpec(bc0r.shape, lambda t: (0, 0)),
            pl.BlockSpec(wg1.shape, lambda t: (0, 0)),
            pl.BlockSpec(bg1r.shape, lambda t: (0, 0)),
            pl.BlockSpec(wc1.shape, lambda t: (0, 0)),
            pl.BlockSpec(bc1r.shape, lambda t: (0, 0)),
        ],
        out_specs=[
            pl.BlockSpec((1, _NP, _BH), lambda t: (t, 0, 0)),
            pl.BlockSpec((_NP, _BH), lambda t: (0, 0)),
        ],
        out_shape=[
            jax.ShapeDtypeStruct((_T, _NP, _BH), _BF),
            jax.ShapeDtypeStruct((_NP, _BH), _BF),
        ],
        scratch_shapes=[
            pltpu.VMEM((_NP, _BH), _F32),
            pltpu.VMEM((_NP, _BH), _F32),
            pltpu.VMEM((_NQT, _QR, _P2), _BF),
            pltpu.VMEM((_NQT, _QR, _P2), _BF),
        ],
        compiler_params=pltpu.CompilerParams(
            dimension_semantics=("arbitrary",),
            vmem_limit_bytes=100 * 1024 * 1024),
    )(x, A_p, gg, gc, wg0, bg0r, wc0, bc0r, wg1, bg1r, wc1, bc1r)

    cur = (out1[:, :_N]
           .astype(_F32)
           .reshape(_T, _N, _B, _H)
           .transpose(0, 2, 1, 3)
           .reshape(_T, _B, _N * _H))
    h0out = (h0fin[:_N]
             .astype(_F32)
             .reshape(_N, _B, _H)
             .transpose(1, 0, 2)
             .reshape(_B, _N * _H))
    return jnp.stack([h0out, cur[-1]], axis=0), cur
